# baseline (speedup 1.0000x reference)
import jax
import jax.numpy as jnp
from jax.experimental import pallas as pl

_S = 8


def _fused_router_kernel(*refs):
    x_refs = refs[:_S]
    w1_ref, b1_ref, w2_ref, b2_ref, o_ref = refs[_S:]
    bk = x_refs[0].shape[1]
    acc = None
    for s in range(_S):
        xb = x_refs[s][...].astype(jnp.bfloat16)
        w1s = w1_ref[s * bk : (s + 1) * bk, :]
        d = jnp.dot(xb, w1s, preferred_element_type=jnp.float32)
        acc = d if acc is None else acc + d
    h = jnp.maximum(acc + b1_ref[...], 0.0)
    o_ref[...] = (
        jnp.dot(h.astype(jnp.bfloat16), w2_ref[...], preferred_element_type=jnp.float32)
        + b2_ref[...]
    )


def kernel(x, W1, b1, W2, b2):
    M, K = x.shape
    H = W1.shape[1]
    G = W2.shape[1]
    BM = 512
    BK = K // _S

    x_specs = [
        pl.BlockSpec((BM, BK), lambda i, s=s: (i, s)) for s in range(_S)
    ]
    return pl.pallas_call(
        _fused_router_kernel,
        grid=(M // BM,),
        in_specs=x_specs
        + [
            pl.BlockSpec((K, H), lambda i: (0, 0)),
            pl.BlockSpec((1, H), lambda i: (0, 0)),
            pl.BlockSpec((H, G), lambda i: (0, 0)),
            pl.BlockSpec((1, G), lambda i: (0, 0)),
        ],
        out_specs=pl.BlockSpec((BM, G), lambda i: (i, 0)),
        out_shape=jax.ShapeDtypeStruct((M, G), jnp.float32),
    )(
        *([x] * _S),
        W1.astype(jnp.bfloat16),
        b1.reshape(1, H),
        W2.astype(jnp.bfloat16),
        b2.reshape(1, G),
    )


# speedup vs baseline: 1.1064x; 1.1064x over previous
import jax
import jax.numpy as jnp
from jax.experimental import pallas as pl

_S = 8


def _fused_router_kernel(*refs):
    x_refs = refs[:_S]
    w1_ref, b1_ref, w2_ref, b2_ref, o_ref = refs[_S:]
    bk = x_refs[0].shape[1]
    acc = None
    for s in range(_S):
        xb = x_refs[s][...].astype(jnp.bfloat16)
        w1s = w1_ref[s * bk : (s + 1) * bk, :]
        d = jnp.dot(xb, w1s, preferred_element_type=jnp.float32)
        acc = d if acc is None else acc + d
    h = jnp.maximum(acc + b1_ref[...], 0.0)
    o_ref[...] = (
        jnp.dot(h.astype(jnp.bfloat16), w2_ref[...], preferred_element_type=jnp.float32)
        + b2_ref[...]
    )


def kernel(x, W1, b1, W2, b2):
    M, K = x.shape
    H = W1.shape[1]
    G = W2.shape[1]
    BM = 1024
    BK = K // _S

    x_specs = [
        pl.BlockSpec((BM, BK), lambda i, s=s: (i, s)) for s in range(_S)
    ]
    return pl.pallas_call(
        _fused_router_kernel,
        grid=(M // BM,),
        in_specs=x_specs
        + [
            pl.BlockSpec((K, H), lambda i: (0, 0)),
            pl.BlockSpec((1, H), lambda i: (0, 0)),
            pl.BlockSpec((H, G), lambda i: (0, 0)),
            pl.BlockSpec((1, G), lambda i: (0, 0)),
        ],
        out_specs=pl.BlockSpec((BM, G), lambda i: (i, 0)),
        out_shape=jax.ShapeDtypeStruct((M, G), jnp.float32),
    )(
        *([x] * _S),
        W1.astype(jnp.bfloat16),
        b1.reshape(1, H),
        W2.astype(jnp.bfloat16),
        b2.reshape(1, G),
    )


# speedup vs baseline: 1.1475x; 1.0371x over previous
import jax
import jax.numpy as jnp
from jax.experimental import pallas as pl


def _fused_router_kernel(x_ref, w1_ref, b1_ref, w2_ref, b2_ref, o_ref):
    h = jnp.dot(x_ref[...], w1_ref[...], preferred_element_type=jnp.float32)
    h = jnp.maximum(h + b1_ref[...], 0.0)
    o_ref[...] = (
        jnp.dot(h, w2_ref[...], preferred_element_type=jnp.float32) + b2_ref[...]
    )


def kernel(x, W1, b1, W2, b2):
    M, K = x.shape
    H = W1.shape[1]
    G = W2.shape[1]
    BM = 1024

    return pl.pallas_call(
        _fused_router_kernel,
        grid=(M // BM,),
        in_specs=[
            pl.BlockSpec((BM, K), lambda i: (i, 0)),
            pl.BlockSpec((K, H), lambda i: (0, 0)),
            pl.BlockSpec((1, H), lambda i: (0, 0)),
            pl.BlockSpec((H, G), lambda i: (0, 0)),
            pl.BlockSpec((1, G), lambda i: (0, 0)),
        ],
        out_specs=pl.BlockSpec((BM, G), lambda i: (i, 0)),
        out_shape=jax.ShapeDtypeStruct((M, G), jnp.float32),
    )(x, W1, b1.reshape(1, H), W2, b2.reshape(1, G))


# speedup vs baseline: 1.2439x; 1.0840x over previous
import jax
import jax.numpy as jnp
from jax.experimental import pallas as pl


def _fused_router_kernel(x_ref, w1_ref, b1_ref, w2t_ref, b2t_ref, o_ref):
    h = jnp.dot(x_ref[...], w1_ref[...], preferred_element_type=jnp.float32)
    h = jnp.maximum(h + b1_ref[...], 0.0)
    s = jnp.dot(h, w2t_ref[...].T, preferred_element_type=jnp.float32)
    o_ref[...] = s.T + b2t_ref[...]


def kernel(x, W1, b1, W2, b2):
    M, K = x.shape
    H = W1.shape[1]
    G = W2.shape[1]
    BM = 1024

    out_t = pl.pallas_call(
        _fused_router_kernel,
        grid=(M // BM,),
        in_specs=[
            pl.BlockSpec((BM, K), lambda i: (i, 0)),
            pl.BlockSpec((K, H), lambda i: (0, 0)),
            pl.BlockSpec((1, H), lambda i: (0, 0)),
            pl.BlockSpec((G, H), lambda i: (0, 0)),
            pl.BlockSpec((G, 1), lambda i: (0, 0)),
        ],
        out_specs=pl.BlockSpec((G, BM), lambda i: (0, i)),
        out_shape=jax.ShapeDtypeStruct((G, M), jnp.float32),
    )(x, W1, b1.reshape(1, H), W2.T, b2.reshape(G, 1))
    return out_t.T


# speedup vs baseline: 1.2460x; 1.0017x over previous
import jax
import jax.numpy as jnp
from jax.experimental import pallas as pl


def _fused_router_kernel(x_ref, w1_ref, b1_ref, w2t_ref, b2_ref, o_ref):
    h = jnp.dot(x_ref[...], w1_ref[...], preferred_element_type=jnp.float32)
    h = jnp.maximum(h + b1_ref[...], 0.0)
    s = jnp.dot(h, w2t_ref[...].T, preferred_element_type=jnp.float32)
    o_ref[...] = (s + b2_ref[...]).T


def kernel(x, W1, b1, W2, b2):
    M, K = x.shape
    H = W1.shape[1]
    G = W2.shape[1]
    BM = 1024

    out_t = pl.pallas_call(
        _fused_router_kernel,
        grid=(M // BM,),
        in_specs=[
            pl.BlockSpec((BM, K), lambda i: (i, 0)),
            pl.BlockSpec((K, H), lambda i: (0, 0)),
            pl.BlockSpec((1, H), lambda i: (0, 0)),
            pl.BlockSpec((G, H), lambda i: (0, 0)),
            pl.BlockSpec((1, G), lambda i: (0, 0)),
        ],
        out_specs=pl.BlockSpec((G, BM), lambda i: (0, i)),
        out_shape=jax.ShapeDtypeStruct((G, M), jnp.float32),
    )(x, W1, b1.reshape(1, H), W2.T, b2.reshape(1, G))
    return out_t.T
